# baseline (speedup 1.0000x reference)
import functools

import jax
import jax.numpy as jnp
from jax import lax
from jax.experimental import pallas as pl
from jax.experimental.pallas import tpu as pltpu
from jax.experimental.pallas import tpu_sc as plsc

_CCHUNK = 4096


def kernel(labels, table):
    B, = labels.shape
    V, D = table.shape

    info = plsc.get_sparse_core_info()
    NC, NS = info.num_cores, info.num_subcores
    NW = NC * NS
    n_rounds = D // NW

    tableT = table.T
    mesh = plsc.VectorSubcoreMesh(core_axis_name="c", subcore_axis_name="s")

    @functools.partial(
        pl.kernel,
        out_type=jax.ShapeDtypeStruct((D, B), jnp.float32),
        mesh=mesh,
        scratch_types=[
            pltpu.VMEM((V,), jnp.float32),
            pltpu.VMEM((B,), jnp.int32),
            pltpu.VMEM((2, _CCHUNK), jnp.float32),
            pltpu.SemaphoreType.DMA,
            pltpu.SemaphoreType.DMA,
            pltpu.SemaphoreType.DMA,
        ],
        compiler_params=pltpu.CompilerParams(
            use_tc_tiling_on_sc=True, needs_layout_passes=False),
    )
    def emb(labels_hbm, tableT_hbm, outT_hbm, slab_v, lab_v, col_v,
            slab_sem, sem0, sem1):
        wid = lax.axis_index("s") * NC + lax.axis_index("c")
        out_sems = (sem0, sem1)
        slab_cp = pltpu.async_copy(tableT_hbm.at[wid], slab_v, slab_sem)
        pltpu.sync_copy(labels_hbm, lab_v)
        writes = [None, None]
        for r in range(n_rounds):
            d = wid + r * NW
            slab_cp.wait()
            for h in range(B // _CCHUNK):
                buf = h % 2
                if writes[buf] is not None:
                    writes[buf].wait()

                @plsc.parallel_loop(0, _CCHUNK, step=16, unroll=4)
                def body(k):
                    idx = lab_v[pl.ds(h * _CCHUNK + k, 16)]
                    col_v[buf, pl.ds(k, 16)] = plsc.load_gather(slab_v, [idx])

                writes[buf] = pltpu.async_copy(
                    col_v.at[buf],
                    outT_hbm.at[d, pl.ds(h * _CCHUNK, _CCHUNK)],
                    out_sems[buf])
            for buf in range(2):
                writes[buf].wait()
                writes[buf] = None
            if r + 1 < n_rounds:
                slab_cp = pltpu.async_copy(
                    tableT_hbm.at[wid + (r + 1) * NW], slab_v, slab_sem)

    return emb(labels, tableT).T


# speedup vs baseline: 1.0040x; 1.0040x over previous
import functools

import jax
import jax.numpy as jnp
from jax import lax
from jax.experimental import pallas as pl
from jax.experimental.pallas import tpu as pltpu
from jax.experimental.pallas import tpu_sc as plsc

_CCHUNK = 4096


def kernel(labels, table):
    B, = labels.shape
    V, D = table.shape

    info = plsc.get_sparse_core_info()
    NC, NS = info.num_cores, info.num_subcores
    NW = NC * NS
    n_rounds = D // NW

    tableT = table.T
    mesh = plsc.VectorSubcoreMesh(core_axis_name="c", subcore_axis_name="s")

    @functools.partial(
        pl.kernel,
        out_type=jax.ShapeDtypeStruct((D, B), jnp.float32),
        mesh=mesh,
        scratch_types=[
            pltpu.VMEM((V,), jnp.float32),
            pltpu.VMEM((B,), jnp.int32),
            pltpu.VMEM((2, _CCHUNK), jnp.float32),
            pltpu.SemaphoreType.DMA,
            pltpu.SemaphoreType.DMA,
            pltpu.SemaphoreType.DMA,
        ],
        compiler_params=pltpu.CompilerParams(
            use_tc_tiling_on_sc=True, needs_layout_passes=False),
    )
    def emb(labels_hbm, tableT_hbm, outT_hbm, slab_v, lab_v, col_v,
            slab_sem, sem0, sem1):
        wid = lax.axis_index("s") * NC + lax.axis_index("c")
        out_sems = (sem0, sem1)
        slab_cp = pltpu.async_copy(tableT_hbm.at[wid], slab_v, slab_sem)
        pltpu.sync_copy(labels_hbm, lab_v)
        writes = [None, None]
        for r in range(n_rounds):
            d = wid + r * NW
            slab_cp.wait()
            for h in range(B // _CCHUNK):
                buf = h % 2
                if writes[buf] is not None:
                    writes[buf].wait()

                @plsc.parallel_loop(0, _CCHUNK, step=16, unroll=8)
                def body(k):
                    idx = lab_v[pl.ds(h * _CCHUNK + k, 16)]
                    col_v[buf, pl.ds(k, 16)] = plsc.load_gather(slab_v, [idx])

                writes[buf] = pltpu.async_copy(
                    col_v.at[buf],
                    outT_hbm.at[d, pl.ds(h * _CCHUNK, _CCHUNK)],
                    out_sems[buf])
            for buf in range(2):
                writes[buf].wait()
                writes[buf] = None
            if r + 1 < n_rounds:
                slab_cp = pltpu.async_copy(
                    tableT_hbm.at[wid + (r + 1) * NW], slab_v, slab_sem)

    return emb(labels, tableT).T
